# baseline (speedup 1.0000x reference)
import functools

import jax
import jax.numpy as jnp
from jax import lax
from jax.experimental import pallas as pl
from jax.experimental.pallas import tpu as pltpu
from jax.experimental.pallas import tpu_sc as plsc

_EMB = 128
_IDX_CHUNK = 128


def _sc_gather(table, idx):
    B = idx.shape[0]
    info = plsc.get_sparse_core_info()
    nw = info.num_cores * info.num_subcores
    bpw = B // nw
    nchunks = bpw // _IDX_CHUNK
    mesh = plsc.VectorSubcoreMesh(core_axis_name="c", subcore_axis_name="s")

    @functools.partial(
        pl.kernel,
        mesh=mesh,
        out_type=jax.ShapeDtypeStruct((B, _EMB), jnp.float32),
        scratch_types=[
            pltpu.VMEM((bpw,), jnp.int32),
            pltpu.VMEM((bpw, _EMB), jnp.float32),
            pltpu.SemaphoreType.DMA,
        ],
    )
    def k(table_hbm, idx_hbm, out_hbm, idx_v, rows_v, sem):
        wid = lax.axis_index("s") * info.num_cores + lax.axis_index("c")
        base = wid * bpw
        pltpu.sync_copy(idx_hbm.at[pl.ds(base, bpw)], idx_v)
        copies = [
            pltpu.async_copy(
                table_hbm.at[idx_v.at[pl.ds(j * _IDX_CHUNK, _IDX_CHUNK)]],
                rows_v.at[pl.ds(j * _IDX_CHUNK, _IDX_CHUNK)],
                sem,
            )
            for j in range(nchunks)
        ]
        for c in copies:
            c.wait()
        pltpu.sync_copy(rows_v, out_hbm.at[pl.ds(base, bpw)])

    return k(table, idx)


def _tc_combine(xg, wt, rate_emb, bm=512):
    B, G = xg.shape

    def body(xg_ref, wt_ref, rate_ref, out_ref):
        xf = xg_ref[...]
        s = jnp.sum(xf, axis=1, keepdims=True)
        g = jnp.dot(xf, wt_ref[...], preferred_element_type=jnp.float32)
        out_ref[:, :_EMB] = rate_ref[...]
        out_ref[:, _EMB:] = g / s

    return pl.pallas_call(
        body,
        grid=(B // bm,),
        in_specs=[
            pl.BlockSpec((bm, G), lambda i: (i, 0)),
            pl.BlockSpec((G, _EMB), lambda i: (0, 0)),
            pl.BlockSpec((bm, _EMB), lambda i: (i, 0)),
        ],
        out_specs=pl.BlockSpec((bm, 2 * _EMB), lambda i: (i, 0)),
        out_shape=jax.ShapeDtypeStruct((B, 2 * _EMB), jnp.float32),
    )(xg, wt, rate_emb)


def kernel(x, embedding_rate, genre_weight):
    rate_idx = x[:, 0]
    xg = x[:, 1:].astype(jnp.float32)
    wt = genre_weight.T
    rate_emb = _sc_gather(embedding_rate, rate_idx)
    return _tc_combine(xg, wt, rate_emb)


# speedup vs baseline: 5.6777x; 5.6777x over previous
import functools

import jax
import jax.numpy as jnp
from jax import lax
from jax.experimental import pallas as pl
from jax.experimental.pallas import tpu as pltpu
from jax.experimental.pallas import tpu_sc as plsc

_EMB = 128
_IDX_CHUNK = 128


def _sc_gather(table, idx):
    B = idx.shape[0]
    V = table.shape[0]
    info = plsc.get_sparse_core_info()
    nw = info.num_cores * info.num_subcores
    bpw = B // nw
    nchunks = bpw // _IDX_CHUNK
    mesh = plsc.VectorSubcoreMesh(core_axis_name="c", subcore_axis_name="s")

    @functools.partial(
        pl.kernel,
        mesh=mesh,
        out_type=jax.ShapeDtypeStruct((B, _EMB), jnp.float32),
        scratch_types=[
            pltpu.VMEM((bpw,), jnp.int32),
            pltpu.VMEM((bpw, _EMB), jnp.float32),
            pltpu.VMEM_SHARED((V, _EMB), jnp.float32),
            pltpu.SemaphoreType.DMA,
        ],
    )
    def k(table_hbm, idx_hbm, out_hbm, idx_v, rows_v, table_sp, sem):
        sid = lax.axis_index("s")
        wid = sid * info.num_cores + lax.axis_index("c")
        base = wid * bpw

        @pl.when(sid == 0)
        def _():
            pltpu.sync_copy(table_hbm, table_sp)

        pltpu.sync_copy(idx_hbm.at[pl.ds(base, bpw)], idx_v)
        plsc.subcore_barrier()
        copies = [
            pltpu.async_copy(
                table_sp.at[idx_v.at[pl.ds(j * _IDX_CHUNK, _IDX_CHUNK)]],
                rows_v.at[pl.ds(j * _IDX_CHUNK, _IDX_CHUNK)],
                sem,
            )
            for j in range(nchunks)
        ]
        for c in copies:
            c.wait()
        pltpu.sync_copy(rows_v, out_hbm.at[pl.ds(base, bpw)])

    return k(table, idx)


def _tc_combine(xg, wt, rate_emb, bm=512):
    B, G = xg.shape

    def body(xg_ref, wt_ref, rate_ref, out_ref):
        xf = xg_ref[...]
        s = jnp.sum(xf, axis=1, keepdims=True)
        g = jnp.dot(xf, wt_ref[...], preferred_element_type=jnp.float32)
        out_ref[:, :_EMB] = rate_ref[...]
        out_ref[:, _EMB:] = g / s

    return pl.pallas_call(
        body,
        grid=(B // bm,),
        in_specs=[
            pl.BlockSpec((bm, G), lambda i: (i, 0)),
            pl.BlockSpec((G, _EMB), lambda i: (0, 0)),
            pl.BlockSpec((bm, _EMB), lambda i: (i, 0)),
        ],
        out_specs=pl.BlockSpec((bm, 2 * _EMB), lambda i: (i, 0)),
        out_shape=jax.ShapeDtypeStruct((B, 2 * _EMB), jnp.float32),
    )(xg, wt, rate_emb)


def kernel(x, embedding_rate, genre_weight):
    rate_idx = x[:, 0]
    xg = x[:, 1:].astype(jnp.float32)
    wt = genre_weight.T
    rate_emb = _sc_gather(embedding_rate, rate_idx)
    return _tc_combine(xg, wt, rate_emb)


# speedup vs baseline: 7.8096x; 1.3755x over previous
import functools

import jax
import jax.numpy as jnp
from jax import lax
from jax.experimental import pallas as pl
from jax.experimental.pallas import tpu as pltpu
from jax.experimental.pallas import tpu_sc as plsc

_EMB = 128
_IDX_CHUNK = 128


def _sc_gather_left(table, idx):
    B = idx.shape[0]
    V = table.shape[0]
    info = plsc.get_sparse_core_info()
    nw = info.num_cores * info.num_subcores
    bpw = B // nw
    nchunks = bpw // _IDX_CHUNK
    mesh = plsc.VectorSubcoreMesh(core_axis_name="c", subcore_axis_name="s")

    @functools.partial(
        pl.kernel,
        mesh=mesh,
        out_type=jax.ShapeDtypeStruct((B, 2 * _EMB), jnp.float32),
        scratch_types=[
            pltpu.VMEM((bpw,), jnp.int32),
            pltpu.VMEM((bpw, _EMB), jnp.float32),
            pltpu.VMEM_SHARED((V, _EMB), jnp.float32),
            pltpu.SemaphoreType.DMA,
        ],
    )
    def k(table_hbm, idx_hbm, out_hbm, idx_v, rows_v, table_sp, sem):
        sid = lax.axis_index("s")
        wid = sid * info.num_cores + lax.axis_index("c")
        base = wid * bpw

        @pl.when(sid == 0)
        def _():
            pltpu.sync_copy(table_hbm, table_sp)

        pltpu.sync_copy(idx_hbm.at[pl.ds(base, bpw)], idx_v)
        plsc.subcore_barrier()
        copies = [
            pltpu.async_copy(
                table_sp.at[idx_v.at[pl.ds(j * _IDX_CHUNK, _IDX_CHUNK)]],
                rows_v.at[pl.ds(j * _IDX_CHUNK, _IDX_CHUNK)],
                sem,
            )
            for j in range(nchunks)
        ]
        for c in copies:
            c.wait()
        pltpu.sync_copy(rows_v, out_hbm.at[pl.ds(base, bpw), pl.ds(0, _EMB)])

    return k(table, idx)


def _tc_genre_right(x, wt_pad, sc_out, bm=512):
    B, C = x.shape

    def body(x_ref, wt_ref, sc_ref, out_ref):
        del sc_ref
        xf = x_ref[...].astype(jnp.float32)
        mask = lax.broadcasted_iota(jnp.int32, (bm, C), 1) > 0
        s = jnp.sum(jnp.where(mask, xf, 0.0), axis=1, keepdims=True)
        g = jnp.dot(xf, wt_ref[...], preferred_element_type=jnp.float32)
        out_ref[...] = g / s

    return pl.pallas_call(
        body,
        grid=(B // bm,),
        in_specs=[
            pl.BlockSpec((bm, C), lambda i: (i, 0)),
            pl.BlockSpec((C, _EMB), lambda i: (0, 0)),
            pl.BlockSpec(memory_space=pl.ANY),
        ],
        out_specs=pl.BlockSpec((bm, _EMB), lambda i: (i, 1)),
        out_shape=jax.ShapeDtypeStruct((B, 2 * _EMB), jnp.float32),
        input_output_aliases={2: 0},
    )(x, wt_pad, sc_out)


def kernel(x, embedding_rate, genre_weight):
    rate_idx = x[:, 0]
    wt_pad = jnp.concatenate(
        [jnp.zeros((1, _EMB), jnp.float32), genre_weight.T], axis=0
    )
    sc_out = _sc_gather_left(embedding_rate, rate_idx)
    return _tc_genre_right(x, wt_pad, sc_out)


# speedup vs baseline: 10.2010x; 1.3062x over previous
import functools

import jax
import jax.numpy as jnp
from jax import lax
from jax.experimental import pallas as pl
from jax.experimental.pallas import tpu as pltpu
from jax.experimental.pallas import tpu_sc as plsc

_EMB = 128
_IDX_CHUNK = 128


def _sc_gather_left(table, idx):
    B = idx.shape[0]
    V = table.shape[0]
    info = plsc.get_sparse_core_info()
    nw = info.num_cores * info.num_subcores
    bpw = B // nw
    nchunks = bpw // _IDX_CHUNK
    mesh = plsc.VectorSubcoreMesh(core_axis_name="c", subcore_axis_name="s")

    @functools.partial(
        pl.kernel,
        mesh=mesh,
        out_type=jax.ShapeDtypeStruct((B, 2 * _EMB), jnp.float32),
        scratch_types=[
            pltpu.VMEM((bpw,), jnp.int32),
            pltpu.VMEM((bpw, _EMB), jnp.float32),
            pltpu.VMEM_SHARED((V, _EMB), jnp.float32),
            pltpu.SemaphoreType.DMA,
        ],
    )
    def k(table_hbm, idx_hbm, out_hbm, idx_v, rows_v, table_sp, sem):
        sid = lax.axis_index("s")
        wid = sid * info.num_cores + lax.axis_index("c")
        base = wid * bpw

        @pl.when(sid == 0)
        def _():
            pltpu.sync_copy(table_hbm, table_sp)

        pltpu.sync_copy(idx_hbm.at[pl.ds(base, bpw)], idx_v)
        plsc.subcore_barrier()
        copies = [
            pltpu.async_copy(
                table_sp.at[idx_v.at[pl.ds(j * _IDX_CHUNK, _IDX_CHUNK)]],
                rows_v.at[pl.ds(j * _IDX_CHUNK, _IDX_CHUNK)],
                sem,
            )
            for j in range(nchunks)
        ]
        for c in copies:
            c.wait()
        pltpu.sync_copy(rows_v, out_hbm.at[pl.ds(base, bpw), pl.ds(0, _EMB)])

    return k(table, idx)


def _tc_genre_right(x, wt_pad, sc_out, bm=2048):
    B, C = x.shape

    def body(x_ref, wt_ref, sc_ref, out_ref):
        del sc_ref
        xf = x_ref[...].astype(jnp.float32)
        mask = lax.broadcasted_iota(jnp.int32, (bm, C), 1) > 0
        s = jnp.sum(jnp.where(mask, xf, 0.0), axis=1, keepdims=True)
        g = jnp.dot(xf, wt_ref[...], preferred_element_type=jnp.float32)
        out_ref[...] = g / s

    return pl.pallas_call(
        body,
        grid=(B // bm,),
        in_specs=[
            pl.BlockSpec((bm, C), lambda i: (i, 0)),
            pl.BlockSpec((C, _EMB), lambda i: (0, 0)),
            pl.BlockSpec(memory_space=pl.ANY),
        ],
        out_specs=pl.BlockSpec((bm, _EMB), lambda i: (i, 1)),
        out_shape=jax.ShapeDtypeStruct((B, 2 * _EMB), jnp.float32),
        input_output_aliases={2: 0},
    )(x, wt_pad, sc_out)


def kernel(x, embedding_rate, genre_weight):
    wt_pad = jnp.concatenate(
        [jnp.zeros((1, _EMB), jnp.float32), genre_weight.T], axis=0
    )
    sc_out = _sc_gather_left(embedding_rate, x[:, 0])
    return _tc_genre_right(x, wt_pad, sc_out)


# speedup vs baseline: 10.8931x; 1.0678x over previous
import functools

import jax
import jax.numpy as jnp
from jax import lax
from jax.experimental import pallas as pl
from jax.experimental.pallas import tpu as pltpu
from jax.experimental.pallas import tpu_sc as plsc

_EMB = 128
_IDX_CHUNK = 128


def _sc_gather_left(table, idx):
    B = idx.shape[0]
    V = table.shape[0]
    info = plsc.get_sparse_core_info()
    nw = info.num_cores * info.num_subcores
    bpw = B // nw
    nchunks = bpw // _IDX_CHUNK
    mesh = plsc.VectorSubcoreMesh(core_axis_name="c", subcore_axis_name="s")

    @functools.partial(
        pl.kernel,
        mesh=mesh,
        out_type=jax.ShapeDtypeStruct((B, 2 * _EMB), jnp.float32),
        scratch_types=[
            pltpu.VMEM((bpw,), jnp.int32),
            pltpu.VMEM((bpw, _EMB), jnp.float32),
            pltpu.VMEM_SHARED((V, _EMB), jnp.float32),
            pltpu.SemaphoreType.DMA,
        ],
    )
    def k(table_hbm, idx_hbm, out_hbm, idx_v, rows_v, table_sp, sem):
        sid = lax.axis_index("s")
        wid = sid * info.num_cores + lax.axis_index("c")
        base = wid * bpw

        @pl.when(sid == 0)
        def _():
            pltpu.sync_copy(table_hbm, table_sp)

        pltpu.sync_copy(idx_hbm.at[pl.ds(base, bpw)], idx_v)
        plsc.subcore_barrier()
        copies = [
            pltpu.async_copy(
                table_sp.at[idx_v.at[pl.ds(j * _IDX_CHUNK, _IDX_CHUNK)]],
                rows_v.at[pl.ds(j * _IDX_CHUNK, _IDX_CHUNK)],
                sem,
            )
            for j in range(nchunks)
        ]
        for c in copies:
            c.wait()
        pltpu.sync_copy(rows_v, out_hbm.at[pl.ds(base, bpw), pl.ds(0, _EMB)])

    return k(table, idx)


def _tc_genre_right(x, wt_pad, sc_out, bm=4096):
    B, C = x.shape

    def body(x_ref, wt_ref, sc_ref, out_ref):
        del sc_ref
        xf = x_ref[...].astype(jnp.float32)
        mask = lax.broadcasted_iota(jnp.int32, (bm, C), 1) > 0
        s = jnp.sum(jnp.where(mask, xf, 0.0), axis=1, keepdims=True)
        g = jnp.dot(xf, wt_ref[...], preferred_element_type=jnp.float32)
        out_ref[...] = g / s

    return pl.pallas_call(
        body,
        grid=(B // bm,),
        in_specs=[
            pl.BlockSpec((bm, C), lambda i: (i, 0)),
            pl.BlockSpec((C, _EMB), lambda i: (0, 0)),
            pl.BlockSpec(memory_space=pl.ANY),
        ],
        out_specs=pl.BlockSpec((bm, _EMB), lambda i: (i, 1)),
        out_shape=jax.ShapeDtypeStruct((B, 2 * _EMB), jnp.float32),
        input_output_aliases={2: 0},
    )(x, wt_pad, sc_out)


def kernel(x, embedding_rate, genre_weight):
    wt_pad = jnp.concatenate(
        [jnp.zeros((1, _EMB), jnp.float32), genre_weight.T], axis=0
    )
    sc_out = _sc_gather_left(embedding_rate, x[:, 0])
    return _tc_genre_right(x, wt_pad, sc_out)


# speedup vs baseline: 11.2785x; 1.0354x over previous
import functools

import jax
import jax.numpy as jnp
from jax import lax
from jax.experimental import pallas as pl
from jax.experimental.pallas import tpu as pltpu
from jax.experimental.pallas import tpu_sc as plsc

_EMB = 128
_IDX_CHUNK = 128


def _sc_gather_left(table, idx):
    B = idx.shape[0]
    V = table.shape[0]
    info = plsc.get_sparse_core_info()
    nw = info.num_cores * info.num_subcores
    bpw = B // nw
    nchunks = bpw // _IDX_CHUNK
    mesh = plsc.VectorSubcoreMesh(core_axis_name="c", subcore_axis_name="s")

    @functools.partial(
        pl.kernel,
        mesh=mesh,
        out_type=jax.ShapeDtypeStruct((B, 2 * _EMB), jnp.float32),
        scratch_types=[
            pltpu.VMEM((bpw,), jnp.int32),
            pltpu.VMEM((bpw, _EMB), jnp.float32),
            pltpu.VMEM_SHARED((V, _EMB), jnp.float32),
            pltpu.SemaphoreType.DMA,
        ],
    )
    def k(table_hbm, idx_hbm, out_hbm, idx_v, rows_v, table_sp, sem):
        sid = lax.axis_index("s")
        wid = sid * info.num_cores + lax.axis_index("c")
        base = wid * bpw

        @pl.when(sid == 0)
        def _():
            pltpu.sync_copy(table_hbm, table_sp)

        pltpu.sync_copy(idx_hbm.at[pl.ds(base, bpw)], idx_v)
        plsc.subcore_barrier()
        copies = [
            pltpu.async_copy(
                table_sp.at[idx_v.at[pl.ds(j * _IDX_CHUNK, _IDX_CHUNK)]],
                rows_v.at[pl.ds(j * _IDX_CHUNK, _IDX_CHUNK)],
                sem,
            )
            for j in range(nchunks)
        ]
        for c in copies:
            c.wait()
        pltpu.sync_copy(rows_v, out_hbm.at[pl.ds(base, bpw), pl.ds(0, _EMB)])

    return k(table, idx)


def _tc_genre_right(x, wt_pad, sc_out, bm=8192):
    B, C = x.shape

    def body(x_ref, wt_ref, sc_ref, out_ref):
        del sc_ref
        xf = x_ref[...].astype(jnp.float32)
        mask = lax.broadcasted_iota(jnp.int32, (bm, C), 1) > 0
        s = jnp.sum(jnp.where(mask, xf, 0.0), axis=1, keepdims=True)
        g = jnp.dot(xf, wt_ref[...], preferred_element_type=jnp.float32)
        out_ref[...] = g / s

    return pl.pallas_call(
        body,
        grid=(B // bm,),
        in_specs=[
            pl.BlockSpec((bm, C), lambda i: (i, 0)),
            pl.BlockSpec((C, _EMB), lambda i: (0, 0)),
            pl.BlockSpec(memory_space=pl.ANY),
        ],
        out_specs=pl.BlockSpec((bm, _EMB), lambda i: (i, 1)),
        out_shape=jax.ShapeDtypeStruct((B, 2 * _EMB), jnp.float32),
        input_output_aliases={2: 0},
    )(x, wt_pad, sc_out)


def kernel(x, embedding_rate, genre_weight):
    wt_pad = jnp.concatenate(
        [jnp.zeros((1, _EMB), jnp.float32), genre_weight.T], axis=0
    )
    sc_out = _sc_gather_left(embedding_rate, x[:, 0])
    return _tc_genre_right(x, wt_pad, sc_out)


# speedup vs baseline: 11.5426x; 1.0234x over previous
import functools

import jax
import jax.numpy as jnp
from jax import lax
from jax.experimental import pallas as pl
from jax.experimental.pallas import tpu as pltpu
from jax.experimental.pallas import tpu_sc as plsc

_EMB = 128
_IDX_CHUNK = 128


def _sc_gather_left(table, idx):
    B = idx.shape[0]
    V = table.shape[0]
    info = plsc.get_sparse_core_info()
    nw = info.num_cores * info.num_subcores
    bpw = B // nw
    nchunks = bpw // _IDX_CHUNK
    mesh = plsc.VectorSubcoreMesh(core_axis_name="c", subcore_axis_name="s")

    @functools.partial(
        pl.kernel,
        mesh=mesh,
        out_type=jax.ShapeDtypeStruct((B, 2 * _EMB), jnp.float32),
        scratch_types=[
            pltpu.VMEM((bpw,), jnp.int32),
            pltpu.VMEM((bpw, _EMB), jnp.float32),
            pltpu.VMEM_SHARED((V, _EMB), jnp.float32),
            pltpu.SemaphoreType.DMA,
            pltpu.SemaphoreType.DMA,
        ],
    )
    def k(table_hbm, idx_hbm, out_hbm, idx_v, rows_v, table_sp, sem, wsem):
        sid = lax.axis_index("s")
        wid = sid * info.num_cores + lax.axis_index("c")
        base = wid * bpw

        @pl.when(sid == 0)
        def _():
            pltpu.sync_copy(table_hbm, table_sp)

        pltpu.sync_copy(idx_hbm.at[pl.ds(base, bpw)], idx_v)
        plsc.subcore_barrier()
        gathers = [
            pltpu.async_copy(
                table_sp.at[idx_v.at[pl.ds(j * _IDX_CHUNK, _IDX_CHUNK)]],
                rows_v.at[pl.ds(j * _IDX_CHUNK, _IDX_CHUNK)],
                sem,
            )
            for j in range(nchunks)
        ]
        writes = []
        for j, g in enumerate(gathers):
            g.wait()
            writes.append(
                pltpu.async_copy(
                    rows_v.at[pl.ds(j * _IDX_CHUNK, _IDX_CHUNK)],
                    out_hbm.at[
                        pl.ds(base + j * _IDX_CHUNK, _IDX_CHUNK), pl.ds(0, _EMB)
                    ],
                    wsem,
                )
            )
        for w in writes:
            w.wait()

    return k(table, idx)


def _tc_genre_right(x, wt_pad, sc_out, bm=8192):
    B, C = x.shape

    def body(x_ref, wt_ref, sc_ref, out_ref):
        del sc_ref
        xf = x_ref[...].astype(jnp.float32)
        mask = lax.broadcasted_iota(jnp.int32, (bm, C), 1) > 0
        s = jnp.sum(jnp.where(mask, xf, 0.0), axis=1, keepdims=True)
        g = jnp.dot(xf, wt_ref[...], preferred_element_type=jnp.float32)
        out_ref[...] = g / s

    return pl.pallas_call(
        body,
        grid=(B // bm,),
        in_specs=[
            pl.BlockSpec((bm, C), lambda i: (i, 0)),
            pl.BlockSpec((C, _EMB), lambda i: (0, 0)),
            pl.BlockSpec(memory_space=pl.ANY),
        ],
        out_specs=pl.BlockSpec((bm, _EMB), lambda i: (i, 1)),
        out_shape=jax.ShapeDtypeStruct((B, 2 * _EMB), jnp.float32),
        input_output_aliases={2: 0},
    )(x, wt_pad, sc_out)


def kernel(x, embedding_rate, genre_weight):
    wt_pad = jnp.concatenate(
        [jnp.zeros((1, _EMB), jnp.float32), genre_weight.T], axis=0
    )
    sc_out = _sc_gather_left(embedding_rate, x[:, 0])
    return _tc_genre_right(x, wt_pad, sc_out)


# speedup vs baseline: 11.6208x; 1.0068x over previous
import functools

import jax
import jax.numpy as jnp
from jax import lax
from jax.experimental import pallas as pl
from jax.experimental.pallas import tpu as pltpu
from jax.experimental.pallas import tpu_sc as plsc

_EMB = 128
_IDX_CHUNK = 128


def _sc_gather_left(table, idx):
    B = idx.shape[0]
    V = table.shape[0]
    info = plsc.get_sparse_core_info()
    nw = info.num_cores * info.num_subcores
    bpw = B // nw
    nchunks = bpw // _IDX_CHUNK
    mesh = plsc.VectorSubcoreMesh(core_axis_name="c", subcore_axis_name="s")

    @functools.partial(
        pl.kernel,
        mesh=mesh,
        out_type=jax.ShapeDtypeStruct((B, 2 * _EMB), jnp.float32),
        scratch_types=[
            pltpu.VMEM((bpw,), jnp.int32),
            pltpu.VMEM((bpw, _EMB), jnp.float32),
            pltpu.VMEM_SHARED((V, _EMB), jnp.float32),
            pltpu.SemaphoreType.DMA,
            pltpu.SemaphoreType.DMA,
        ],
    )
    def k(table_hbm, idx_hbm, out_hbm, idx_v, rows_v, table_sp, sem, wsem):
        sid = lax.axis_index("s")
        wid = sid * info.num_cores + lax.axis_index("c")
        base = wid * bpw

        ns = info.num_subcores
        stripe = -(-V // ns) // 8 * 8
        nfull = V // stripe
        rem = V - nfull * stripe

        @pl.when(sid < nfull)
        def _():
            pltpu.sync_copy(
                table_hbm.at[pl.ds(sid * stripe, stripe)],
                table_sp.at[pl.ds(sid * stripe, stripe)],
            )

        if rem:

            @pl.when(sid == nfull)
            def _():
                pltpu.sync_copy(
                    table_hbm.at[pl.ds(nfull * stripe, rem)],
                    table_sp.at[pl.ds(nfull * stripe, rem)],
                )

        pltpu.sync_copy(idx_hbm.at[pl.ds(base, bpw)], idx_v)
        plsc.subcore_barrier()
        gathers = [
            pltpu.async_copy(
                table_sp.at[idx_v.at[pl.ds(j * _IDX_CHUNK, _IDX_CHUNK)]],
                rows_v.at[pl.ds(j * _IDX_CHUNK, _IDX_CHUNK)],
                sem,
            )
            for j in range(nchunks)
        ]
        writes = []
        for j, g in enumerate(gathers):
            g.wait()
            writes.append(
                pltpu.async_copy(
                    rows_v.at[pl.ds(j * _IDX_CHUNK, _IDX_CHUNK)],
                    out_hbm.at[
                        pl.ds(base + j * _IDX_CHUNK, _IDX_CHUNK), pl.ds(0, _EMB)
                    ],
                    wsem,
                )
            )
        for w in writes:
            w.wait()

    return k(table, idx)


def _tc_genre_right(x, wt_pad, sc_out, bm=8192):
    B, C = x.shape

    def body(x_ref, wt_ref, sc_ref, out_ref):
        del sc_ref
        xf = x_ref[...].astype(jnp.float32)
        mask = lax.broadcasted_iota(jnp.int32, (bm, C), 1) > 0
        s = jnp.sum(jnp.where(mask, xf, 0.0), axis=1, keepdims=True)
        g = jnp.dot(xf, wt_ref[...], preferred_element_type=jnp.float32)
        out_ref[...] = g / s

    return pl.pallas_call(
        body,
        grid=(B // bm,),
        in_specs=[
            pl.BlockSpec((bm, C), lambda i: (i, 0)),
            pl.BlockSpec((C, _EMB), lambda i: (0, 0)),
            pl.BlockSpec(memory_space=pl.ANY),
        ],
        out_specs=pl.BlockSpec((bm, _EMB), lambda i: (i, 1)),
        out_shape=jax.ShapeDtypeStruct((B, 2 * _EMB), jnp.float32),
        input_output_aliases={2: 0},
    )(x, wt_pad, sc_out)


def kernel(x, embedding_rate, genre_weight):
    wt_pad = jnp.concatenate(
        [jnp.zeros((1, _EMB), jnp.float32), genre_weight.T], axis=0
    )
    sc_out = _sc_gather_left(embedding_rate, x[:, 0])
    return _tc_genre_right(x, wt_pad, sc_out)
